# baseline (speedup 1.0000x reference)
import functools

import jax
import jax.numpy as jnp
from jax import lax
from jax.experimental import pallas as pl
from jax.experimental.pallas import tpu as pltpu
from jax.experimental.pallas import tpu_sc as plsc

N = 10000
E = 320000
D = 128
G = 64
NC = 2
NS = 16
NW = NC * NS
WIN = 128
NWIN = E // WIN
WIN_BASE = NWIN // NW
WIN_REM = NWIN % NW
ROWBLK = 640
NZWIN = (N + WIN - 1) // WIN
ZWPT = 5

_mesh = functools.partial(
    plsc.VectorSubcoreMesh, core_axis_name="c", subcore_axis_name="s")


def _wid():
  return lax.axis_index("s") * NC + lax.axis_index("c")


def _vconsts():
  lane = lax.iota(jnp.int32, 16)
  zero16 = plsc.bitcast(lane * 0, jnp.float32)
  one16 = plsc.bitcast(lane * 0 + 0x3F800000, jnp.float32)
  return lane, zero16, one16


def _sload(ref, i):
  return ref[pl.ds(i, 16)][0]


@functools.partial(
    pl.kernel,
    out_type=jax.ShapeDtypeStruct((NC * N,), jnp.float32),
    mesh=_mesh(),
    scratch_types=[
        pltpu.VMEM((WIN,), jnp.int32),
        pltpu.VMEM((WIN,), jnp.int32),
        pltpu.VMEM((WIN,), jnp.float32),
        pltpu.VMEM((WIN,), jnp.float32),
        pltpu.VMEM((ROWBLK,), jnp.float32),
        pltpu.VMEM_SHARED((N,), jnp.float32),
        pltpu.SemaphoreType.DMA,
        pltpu.SemaphoreType.DMA,
    ],
)
def _deg_kernel(dst2d, iidx2d, ones_hbm, zer_hbm, out,
                didxA, didxB, ones_v, zer_v, stage, acc, semA, semB):
  cid = lax.axis_index("c")
  sid = lax.axis_index("s")
  wid = _wid()
  pltpu.sync_copy(ones_hbm, ones_v)
  pltpu.sync_copy(zer_hbm, zer_v)

  @pl.loop(0, ZWPT)
  def _(j):
    pltpu.sync_copy(iidx2d.at[sid * ZWPT + j], didxA)
    pltpu.sync_copy(zer_v, acc.at[didxA])

  plsc.subcore_barrier()

  base = wid * WIN_BASE + jnp.minimum(wid, WIN_REM)
  nwin = WIN_BASE + jnp.where(wid < WIN_REM, 1, 0)

  pltpu.sync_copy(dst2d.at[base], didxA)

  @pl.when(nwin > 1)
  def _():
    pltpu.async_copy(dst2d.at[base + 1], didxB, semB)

  @pl.loop(0, nwin, step=2)
  def _(w):
    pltpu.sync_copy(ones_v, acc.at[didxA], add=True)

    @pl.when(w + 2 < nwin)
    def _():
      pltpu.async_copy(dst2d.at[base + w + 2], didxA, semA)

    @pl.when(w + 1 < nwin)
    def _():
      pltpu.make_async_copy(dst2d.at[base + w + 1], didxB, semB).wait()
      pltpu.sync_copy(ones_v, acc.at[didxB], add=True)

      @pl.when(w + 3 < nwin)
      def _():
        pltpu.async_copy(dst2d.at[base + w + 3], didxB, semB)

    @pl.when(w + 2 < nwin)
    def _():
      pltpu.make_async_copy(dst2d.at[base + w + 2], didxA, semA).wait()

  plsc.subcore_barrier()
  st = jnp.minimum(sid * ROWBLK, N - ROWBLK)
  off = pl.multiple_of(cid * N + st, 16)
  pltpu.sync_copy(acc.at[pl.ds(st, ROWBLK)], stage)
  pltpu.sync_copy(stage, out.at[pl.ds(off, ROWBLK)])


@functools.partial(
    pl.kernel,
    out_type=jax.ShapeDtypeStruct((NC, N, D), jnp.float32),
    mesh=_mesh(),
    scratch_types=[
        pltpu.VMEM((WIN,), jnp.int32),
        pltpu.VMEM((WIN,), jnp.int32),
        pltpu.VMEM((WIN,), jnp.int32),
        pltpu.VMEM((WIN,), jnp.int32),
        pltpu.VMEM((WIN, D), jnp.float32),
        pltpu.VMEM((WIN, D), jnp.float32),
        pltpu.VMEM_SHARED((N, D), jnp.float32),
        pltpu.SemaphoreType.DMA,
        pltpu.SemaphoreType.DMA,
        pltpu.SemaphoreType.DMA,
        pltpu.SemaphoreType.DMA,
    ],
)
def _scatter_kernel(u_hbm, src2d, dst2d, iidx2d, zrows_hbm, out,
                    sidxA, didxA, sidxB, didxB, rowsA, rowsB, sacc,
                    semGA, semGB, semIA, semIB):
  cid = lax.axis_index("c")
  sid = lax.axis_index("s")
  wid = _wid()

  pltpu.sync_copy(iidx2d.at[sid * ZWPT], sidxA)
  pltpu.async_copy(zrows_hbm.at[sidxA], rowsA, semGA).wait()

  @pl.loop(0, ZWPT)
  def _(j):
    pltpu.sync_copy(iidx2d.at[sid * ZWPT + j], sidxA)
    pltpu.sync_copy(rowsA, sacc.at[sidxA])

  plsc.subcore_barrier()

  base = wid * WIN_BASE + jnp.minimum(wid, WIN_REM)
  nwin = WIN_BASE + jnp.where(wid < WIN_REM, 1, 0)

  def start_idx(win, si, di, sem):
    pltpu.async_copy(src2d.at[win], si, sem)
    pltpu.async_copy(dst2d.at[win], di, sem)

  def wait_idx(win, si, di, sem):
    pltpu.make_async_copy(src2d.at[win], si, sem).wait()
    pltpu.make_async_copy(dst2d.at[win], di, sem).wait()

  def start_g(si, rows, sem):
    pltpu.async_copy(u_hbm.at[si], rows, sem)

  def wait_g(si, rows, sem):
    pltpu.make_async_copy(u_hbm.at[si], rows, sem).wait()

  pltpu.sync_copy(src2d.at[base], sidxA)
  pltpu.sync_copy(dst2d.at[base], didxA)
  start_g(sidxA, rowsA, semGA)

  @pl.when(nwin > 1)
  def _():
    start_idx(base + 1, sidxB, didxB, semIB)

  @pl.loop(0, nwin, step=2)
  def _(w):
    wait_g(sidxA, rowsA, semGA)

    @pl.when(w + 1 < nwin)
    def _():
      wait_idx(base + w + 1, sidxB, didxB, semIB)
      start_g(sidxB, rowsB, semGB)

    pltpu.sync_copy(rowsA, sacc.at[didxA], add=True)

    @pl.when(w + 2 < nwin)
    def _():
      start_idx(base + w + 2, sidxA, didxA, semIA)

    @pl.when(w + 1 < nwin)
    def _():
      wait_g(sidxB, rowsB, semGB)

      @pl.when(w + 2 < nwin)
      def _():
        wait_idx(base + w + 2, sidxA, didxA, semIA)
        start_g(sidxA, rowsA, semGA)

      pltpu.sync_copy(rowsB, sacc.at[didxB], add=True)

      @pl.when(w + 3 < nwin)
      def _():
        start_idx(base + w + 3, sidxB, didxB, semIB)

  plsc.subcore_barrier()
  st = jnp.minimum(sid * ROWBLK, N - ROWBLK)
  pltpu.sync_copy(sacc.at[pl.ds(st, ROWBLK)], out.at[cid, pl.ds(st, ROWBLK)])


NPAD = N + 16


@functools.partial(
    pl.kernel,
    out_type=jax.ShapeDtypeStruct((G * D,), jnp.float32),
    mesh=_mesh(),
    scratch_types=[
        pltpu.VMEM((NPAD,), jnp.int32),
        pltpu.VMEM((NPAD,), jnp.float32),
        pltpu.VMEM((NPAD,), jnp.int32),
        pltpu.VMEM((WIN,), jnp.int32),
        pltpu.VMEM((WIN, D), jnp.float32),
        pltpu.VMEM((WIN, D), jnp.float32),
        pltpu.VMEM((D,), jnp.float32),
        pltpu.VMEM((D,), jnp.float32),
        pltpu.VMEM((D,), jnp.float32),
        pltpu.SemaphoreType.DMA,
    ],
)
def _pool_kernel(batch_hbm, score_hbm, hp_hbm, iidx2d, zer_hbm,
                 wlin_hbm, blin_hbm, out,
                 bb, sb, fl, ridx, hb, wl, ab, ob, bl, sem):
  wid = _wid()
  lane, _, _ = _vconsts()
  pltpu.sync_copy(batch_hbm, bb.at[pl.ds(0, N)])
  pltpu.sync_copy(score_hbm, sb.at[pl.ds(0, N)])
  pltpu.sync_copy(blin_hbm, bl)
  bb[pl.ds(N, 16)] = lane * 0 + G
  pltpu.sync_copy(zer_hbm.at[pl.ds(0, 16)], sb.at[pl.ds(N, 16)])
  for v in range(D // 16):
    ridx[pl.ds(v * 16, 16)] = lane + v * 16
  pltpu.async_copy(wlin_hbm.at[ridx], wl, sem).wait()

  def lower_bound(val):
    def body(_, lohi):
      lo, hi = lohi
      mid = (lo + hi) // 2
      p = _sload(bb, mid) < val
      return jnp.where(p, mid + 1, lo), jnp.where(p, hi, mid)
    lo, _ = lax.fori_loop(0, 14, body, (jnp.int32(0), jnp.int32(N)))
    return lo

  for gg in range(2):
    g = wid * 2 + gg
    st = lower_bound(g)
    en = lower_bound(g + 1)
    c = en - st
    kk = (c + 1) // 2
    pltpu.sync_copy(zer_hbm, ab)

    @pl.when(c > 0)
    def _():
      nchunk = (c + 15) // 16

      @pl.loop(0, nchunk)
      def _(ic):
        ibase = st + ic * 16
        iv = ibase + lane
        si = sb[pl.ds(ibase, 16)]

        def jbody(j, cnt):
          sj = _sload(sb, j)
          hit = (sj > si) | ((sj == si) & (j < iv))
          return cnt + jnp.where(hit, 1, 0)

        cnt = lax.fori_loop(st, en, jbody, lane * 0)
        selv = (cnt < kk) & (iv < en)
        fl[pl.ds(ic * 16, 16)] = jnp.where(selv, 1, 0)

      wb = st // WIN
      nhw = (en - wb * WIN + WIN - 1) // WIN

      @pl.loop(0, nhw)
      def _(wj):
        pltpu.sync_copy(iidx2d.at[wb + wj], ridx)
        pltpu.async_copy(hp_hbm.at[ridx], hb, sem).wait()

        @pl.loop(0, WIN)
        def _(r):
          gr = (wb + wj) * WIN + r
          ok = (gr >= st) & (gr < en)

          @pl.when(ok)
          def _():
            @pl.when(_sload(fl, gr - st) > 0)
            def _():
              for v in range(D // 16):
                sl = pl.ds(v * 16, 16)
                ab[sl] = jnp.maximum(ab[sl], hb[r, sl])

    pltpu.sync_copy(zer_hbm, ob)

    @pl.loop(0, D)
    def _(d):
      pd = _sload(ab, d)

      @pl.when(pd != 0.0)
      def _():
        for v in range(D // 16):
          sl = pl.ds(v * 16, 16)
          ob[sl] = ob[sl] + pd * wl[d, sl]

    for v in range(D // 16):
      sl = pl.ds(v * 16, 16)
      ob[sl] = ob[sl] + bl[sl]

    off = pl.multiple_of(g * D, 16)
    pltpu.sync_copy(ob, out.at[pl.ds(off, D)])


_BLK = 400
_DOT = dict(preferred_element_type=jnp.float32, precision=lax.Precision.HIGHEST)


def _tca_body(x_ref, w_ref, degt_ref, u_ref):
  h = jnp.dot(x_ref[...], w_ref[...], **_DOT)
  deg = degt_ref[:, 0:1] + degt_ref[:, 1:2] + 1.0
  dinv = jnp.where(deg > 0, lax.rsqrt(deg), 0.0)
  u_ref[...] = h * dinv


def _tcb_body(s_ref, u_ref, degt_ref, b_ref, w_ref, score_ref, hp_ref):
  s_sum = s_ref[0] + s_ref[1]
  deg = degt_ref[:, 0:1] + degt_ref[:, 1:2] + 1.0
  dinv = jnp.where(deg > 0, lax.rsqrt(deg), 0.0)
  htot = dinv * (s_sum + u_ref[...]) + b_ref[...]
  w = w_ref[...]
  nw = jnp.sqrt(jnp.sum(w * w))
  z = jnp.dot(htot, w, **_DOT) / nw
  sc = jnp.tanh(z)
  score_ref[...] = sc
  hp_ref[...] = jnp.maximum(htot * sc, 0.0)


def kernel(x, edge_index, batch, W_conv, b_conv, w_pool, W_lin, b_lin):
  src2d = edge_index[0].reshape(NWIN, WIN)
  dst2d = edge_index[1].reshape(NWIN, WIN)
  flat = jnp.minimum(jnp.arange((NZWIN + 1) * WIN, dtype=jnp.int32), N - 1)
  iidx2d = jnp.concatenate(
      [flat, jnp.arange(WIN, dtype=jnp.int32)]).reshape(NZWIN + 2, WIN)
  zrows = jnp.zeros((N, D), jnp.float32)
  ones1 = jnp.ones((WIN,), jnp.float32)
  zer1 = jnp.zeros((WIN,), jnp.float32)

  degp = _deg_kernel(dst2d, iidx2d, ones1, zer1).reshape(NC, N)
  degt = degp.T

  u = pl.pallas_call(
      _tca_body,
      grid=(N // _BLK,),
      in_specs=[
          pl.BlockSpec((_BLK, D), lambda i: (i, 0)),
          pl.BlockSpec((D, D), lambda i: (0, 0)),
          pl.BlockSpec((_BLK, 2), lambda i: (i, 0)),
      ],
      out_specs=pl.BlockSpec((_BLK, D), lambda i: (i, 0)),
      out_shape=jax.ShapeDtypeStruct((N, D), jnp.float32),
  )(x, W_conv, degt)

  s_partials = _scatter_kernel(u, src2d, dst2d, iidx2d, zrows)

  score2, hp = pl.pallas_call(
      _tcb_body,
      grid=(N // _BLK,),
      in_specs=[
          pl.BlockSpec((NC, _BLK, D), lambda i: (0, i, 0)),
          pl.BlockSpec((_BLK, D), lambda i: (i, 0)),
          pl.BlockSpec((_BLK, 2), lambda i: (i, 0)),
          pl.BlockSpec((1, D), lambda i: (0, 0)),
          pl.BlockSpec((D, 1), lambda i: (0, 0)),
      ],
      out_specs=[
          pl.BlockSpec((_BLK, 1), lambda i: (i, 0)),
          pl.BlockSpec((_BLK, D), lambda i: (i, 0)),
      ],
      out_shape=[
          jax.ShapeDtypeStruct((N, 1), jnp.float32),
          jax.ShapeDtypeStruct((N, D), jnp.float32),
      ],
  )(s_partials, u, degt, b_conv.reshape(1, D), w_pool.reshape(D, 1))

  out = _pool_kernel(
      batch, score2.reshape(N), hp, iidx2d, zer1, W_lin, b_lin)
  return out.reshape(G, D)


# speedup vs baseline: 1.0543x; 1.0543x over previous
import functools

import jax
import jax.numpy as jnp
from jax import lax
from jax.experimental import pallas as pl
from jax.experimental.pallas import tpu as pltpu
from jax.experimental.pallas import tpu_sc as plsc

N = 10000
E = 320000
D = 128
G = 64
NC = 2
NS = 16
NW = NC * NS
WIN = 128
NWIN = E // WIN
WIN_BASE = NWIN // NW
WIN_REM = NWIN % NW
ROWBLK = 640
NZWIN = (N + WIN - 1) // WIN
ZWPT = 5

_mesh = functools.partial(
    plsc.VectorSubcoreMesh, core_axis_name="c", subcore_axis_name="s")


def _wid():
  return lax.axis_index("s") * NC + lax.axis_index("c")


def _vconsts():
  lane = lax.iota(jnp.int32, 16)
  zero16 = plsc.bitcast(lane * 0, jnp.float32)
  one16 = plsc.bitcast(lane * 0 + 0x3F800000, jnp.float32)
  return lane, zero16, one16


def _sload(ref, i):
  return ref[pl.ds(i, 16)][0]


@functools.partial(
    pl.kernel,
    out_type=jax.ShapeDtypeStruct((NC * N,), jnp.float32),
    mesh=_mesh(),
    scratch_types=[
        pltpu.VMEM((WIN,), jnp.int32),
        pltpu.VMEM((WIN,), jnp.int32),
        pltpu.VMEM((WIN,), jnp.float32),
        pltpu.VMEM((WIN,), jnp.float32),
        pltpu.VMEM((ROWBLK,), jnp.float32),
        pltpu.VMEM_SHARED((N,), jnp.float32),
        pltpu.SemaphoreType.DMA,
        pltpu.SemaphoreType.DMA,
    ],
)
def _deg_kernel(dst2d, iidx2d, ones_hbm, zer_hbm, out,
                didxA, didxB, ones_v, zer_v, stage, acc, semA, semB):
  cid = lax.axis_index("c")
  sid = lax.axis_index("s")
  wid = _wid()
  pltpu.sync_copy(ones_hbm, ones_v)
  pltpu.sync_copy(zer_hbm, zer_v)

  @pl.loop(0, ZWPT)
  def _(j):
    pltpu.sync_copy(iidx2d.at[sid * ZWPT + j], didxA)
    pltpu.sync_copy(zer_v, acc.at[didxA])

  plsc.subcore_barrier()

  base = wid * WIN_BASE + jnp.minimum(wid, WIN_REM)
  nwin = WIN_BASE + jnp.where(wid < WIN_REM, 1, 0)

  pltpu.sync_copy(dst2d.at[base], didxA)

  @pl.when(nwin > 1)
  def _():
    pltpu.async_copy(dst2d.at[base + 1], didxB, semB)

  @pl.loop(0, nwin, step=2)
  def _(w):
    pltpu.sync_copy(ones_v, acc.at[didxA], add=True)

    @pl.when(w + 2 < nwin)
    def _():
      pltpu.async_copy(dst2d.at[base + w + 2], didxA, semA)

    @pl.when(w + 1 < nwin)
    def _():
      pltpu.make_async_copy(dst2d.at[base + w + 1], didxB, semB).wait()
      pltpu.sync_copy(ones_v, acc.at[didxB], add=True)

      @pl.when(w + 3 < nwin)
      def _():
        pltpu.async_copy(dst2d.at[base + w + 3], didxB, semB)

    @pl.when(w + 2 < nwin)
    def _():
      pltpu.make_async_copy(dst2d.at[base + w + 2], didxA, semA).wait()

  plsc.subcore_barrier()
  st = jnp.minimum(sid * ROWBLK, N - ROWBLK)
  off = pl.multiple_of(cid * N + st, 16)
  pltpu.sync_copy(acc.at[pl.ds(st, ROWBLK)], stage)
  pltpu.sync_copy(stage, out.at[pl.ds(off, ROWBLK)])


@functools.partial(
    pl.kernel,
    out_type=jax.ShapeDtypeStruct((NC, N, D), jnp.float32),
    mesh=_mesh(),
    scratch_types=[
        pltpu.VMEM((WIN,), jnp.int32),
        pltpu.VMEM((WIN,), jnp.int32),
        pltpu.VMEM((WIN,), jnp.int32),
        pltpu.VMEM((WIN,), jnp.int32),
        pltpu.VMEM((WIN, D), jnp.float32),
        pltpu.VMEM((WIN, D), jnp.float32),
        pltpu.VMEM_SHARED((N, D), jnp.float32),
        pltpu.SemaphoreType.DMA,
        pltpu.SemaphoreType.DMA,
        pltpu.SemaphoreType.DMA,
        pltpu.SemaphoreType.DMA,
    ],
)
def _scatter_kernel(u_hbm, src2d, dst2d, iidx2d, zrows_hbm, out,
                    sidxA, didxA, sidxB, didxB, rowsA, rowsB, sacc,
                    semGA, semGB, semIA, semIB):
  cid = lax.axis_index("c")
  sid = lax.axis_index("s")
  wid = _wid()

  pltpu.sync_copy(iidx2d.at[sid * ZWPT], sidxA)
  pltpu.async_copy(zrows_hbm.at[sidxA], rowsA, semGA).wait()

  @pl.loop(0, ZWPT)
  def _(j):
    pltpu.sync_copy(iidx2d.at[sid * ZWPT + j], sidxA)
    pltpu.sync_copy(rowsA, sacc.at[sidxA])

  plsc.subcore_barrier()

  base = wid * WIN_BASE + jnp.minimum(wid, WIN_REM)
  nwin = WIN_BASE + jnp.where(wid < WIN_REM, 1, 0)

  def start_idx(win, si, di, sem):
    pltpu.async_copy(src2d.at[win], si, sem)
    pltpu.async_copy(dst2d.at[win], di, sem)

  def wait_idx(win, si, di, sem):
    pltpu.make_async_copy(src2d.at[win], si, sem).wait()
    pltpu.make_async_copy(dst2d.at[win], di, sem).wait()

  def start_g(si, rows, sem):
    pltpu.async_copy(u_hbm.at[si], rows, sem)

  def wait_g(si, rows, sem):
    pltpu.make_async_copy(u_hbm.at[si], rows, sem).wait()

  pltpu.sync_copy(src2d.at[base], sidxA)
  pltpu.sync_copy(dst2d.at[base], didxA)
  start_g(sidxA, rowsA, semGA)

  @pl.when(nwin > 1)
  def _():
    start_idx(base + 1, sidxB, didxB, semIB)

  @pl.loop(0, nwin, step=2)
  def _(w):
    wait_g(sidxA, rowsA, semGA)

    @pl.when(w + 1 < nwin)
    def _():
      wait_idx(base + w + 1, sidxB, didxB, semIB)
      start_g(sidxB, rowsB, semGB)

    pltpu.sync_copy(rowsA, sacc.at[didxA], add=True)

    @pl.when(w + 2 < nwin)
    def _():
      start_idx(base + w + 2, sidxA, didxA, semIA)

    @pl.when(w + 1 < nwin)
    def _():
      wait_g(sidxB, rowsB, semGB)

      @pl.when(w + 2 < nwin)
      def _():
        wait_idx(base + w + 2, sidxA, didxA, semIA)
        start_g(sidxA, rowsA, semGA)

      pltpu.sync_copy(rowsB, sacc.at[didxB], add=True)

      @pl.when(w + 3 < nwin)
      def _():
        start_idx(base + w + 3, sidxB, didxB, semIB)

  plsc.subcore_barrier()
  st = jnp.minimum(sid * ROWBLK, N - ROWBLK)
  pltpu.sync_copy(sacc.at[pl.ds(st, ROWBLK)], out.at[cid, pl.ds(st, ROWBLK)])


NPAD = N + 16


@functools.partial(
    pl.kernel,
    out_type=jax.ShapeDtypeStruct((G * D,), jnp.float32),
    mesh=_mesh(),
    scratch_types=[
        pltpu.VMEM((NPAD,), jnp.int32),
        pltpu.VMEM((NPAD,), jnp.float32),
        pltpu.VMEM((NPAD,), jnp.int32),
        pltpu.VMEM((WIN,), jnp.int32),
        pltpu.VMEM((WIN, D), jnp.float32),
        pltpu.VMEM((D,), jnp.float32),
        pltpu.SemaphoreType.DMA,
    ],
)
def _pool_kernel(batch_hbm, score_hbm, hp_hbm, iidx2d, zer_hbm, out,
                 bb, sb, fl, ridx, hb, ab, sem):
  wid = _wid()
  lane, _, _ = _vconsts()
  pltpu.sync_copy(batch_hbm, bb.at[pl.ds(0, N)])
  pltpu.sync_copy(score_hbm, sb.at[pl.ds(0, N)])
  bb[pl.ds(N, 16)] = lane * 0 + G
  pltpu.sync_copy(zer_hbm.at[pl.ds(0, 16)], sb.at[pl.ds(N, 16)])

  def lower_bound(val):
    def body(_, lohi):
      lo, hi = lohi
      mid = (lo + hi) // 2
      p = _sload(bb, mid) < val
      return jnp.where(p, mid + 1, lo), jnp.where(p, hi, mid)
    lo, _ = lax.fori_loop(0, 14, body, (jnp.int32(0), jnp.int32(N)))
    return lo

  for gg in range(2):
    g = wid * 2 + gg
    st = lower_bound(g)
    en = lower_bound(g + 1)
    c = en - st
    kk = (c + 1) // 2
    pltpu.sync_copy(zer_hbm, ab)

    @pl.when(c > 0)
    def _():
      nchunk = (c + 15) // 16

      @pl.loop(0, nchunk)
      def _(ic):
        ibase = st + ic * 16
        iv = ibase + lane
        si = sb[pl.ds(ibase, 16)]

        def jbody(j, cnt):
          sj = _sload(sb, j)
          hit = (sj > si) | ((sj == si) & (j < iv))
          return cnt + jnp.where(hit, 1, 0)

        cnt = lax.fori_loop(st, en, jbody, lane * 0)
        selv = (cnt < kk) & (iv < en)
        fl[pl.ds(ic * 16, 16)] = jnp.where(selv, 1, 0)

      wb = st // WIN
      nhw = (en - wb * WIN + WIN - 1) // WIN

      @pl.loop(0, nhw)
      def _(wj):
        pltpu.sync_copy(iidx2d.at[wb + wj], ridx)
        pltpu.async_copy(hp_hbm.at[ridx], hb, sem).wait()

        @pl.loop(0, WIN)
        def _(r):
          gr = (wb + wj) * WIN + r
          ok = (gr >= st) & (gr < en)

          @pl.when(ok)
          def _():
            @pl.when(_sload(fl, gr - st) > 0)
            def _():
              for v in range(D // 16):
                sl = pl.ds(v * 16, 16)
                ab[sl] = jnp.maximum(ab[sl], hb[r, sl])

    off = pl.multiple_of(g * D, 16)
    pltpu.sync_copy(ab, out.at[pl.ds(off, D)])


_BLK = 400
_DOT = dict(preferred_element_type=jnp.float32, precision=lax.Precision.HIGHEST)


def _tca_body(x_ref, w_ref, degt_ref, u_ref):
  h = jnp.dot(x_ref[...], w_ref[...], **_DOT)
  deg = degt_ref[:, 0:1] + degt_ref[:, 1:2] + 1.0
  dinv = jnp.where(deg > 0, lax.rsqrt(deg), 0.0)
  u_ref[...] = h * dinv


def _tcc_body(p_ref, wl_ref, bl_ref, o_ref):
  o_ref[...] = jnp.dot(p_ref[...], wl_ref[...], **_DOT) + bl_ref[...]


def _tcb_body(s_ref, u_ref, degt_ref, b_ref, w_ref, score_ref, hp_ref):
  s_sum = s_ref[0] + s_ref[1]
  deg = degt_ref[:, 0:1] + degt_ref[:, 1:2] + 1.0
  dinv = jnp.where(deg > 0, lax.rsqrt(deg), 0.0)
  htot = dinv * (s_sum + u_ref[...]) + b_ref[...]
  w = w_ref[...]
  nw = jnp.sqrt(jnp.sum(w * w))
  z = jnp.dot(htot, w, **_DOT) / nw
  sc = jnp.tanh(z)
  score_ref[...] = sc
  hp_ref[...] = jnp.maximum(htot * sc, 0.0)


def kernel(x, edge_index, batch, W_conv, b_conv, w_pool, W_lin, b_lin):
  src2d = edge_index[0].reshape(NWIN, WIN)
  dst2d = edge_index[1].reshape(NWIN, WIN)
  flat = jnp.minimum(jnp.arange((NZWIN + 1) * WIN, dtype=jnp.int32), N - 1)
  iidx2d = jnp.concatenate(
      [flat, jnp.arange(WIN, dtype=jnp.int32)]).reshape(NZWIN + 2, WIN)
  zrows = jnp.zeros((N, D), jnp.float32)
  ones1 = jnp.ones((WIN,), jnp.float32)
  zer1 = jnp.zeros((WIN,), jnp.float32)

  degp = _deg_kernel(dst2d, iidx2d, ones1, zer1).reshape(NC, N)
  degt = degp.T

  u = pl.pallas_call(
      _tca_body,
      grid=(N // _BLK,),
      in_specs=[
          pl.BlockSpec((_BLK, D), lambda i: (i, 0)),
          pl.BlockSpec((D, D), lambda i: (0, 0)),
          pl.BlockSpec((_BLK, 2), lambda i: (i, 0)),
      ],
      out_specs=pl.BlockSpec((_BLK, D), lambda i: (i, 0)),
      out_shape=jax.ShapeDtypeStruct((N, D), jnp.float32),
  )(x, W_conv, degt)

  s_partials = _scatter_kernel(u, src2d, dst2d, iidx2d, zrows)

  score2, hp = pl.pallas_call(
      _tcb_body,
      grid=(N // _BLK,),
      in_specs=[
          pl.BlockSpec((NC, _BLK, D), lambda i: (0, i, 0)),
          pl.BlockSpec((_BLK, D), lambda i: (i, 0)),
          pl.BlockSpec((_BLK, 2), lambda i: (i, 0)),
          pl.BlockSpec((1, D), lambda i: (0, 0)),
          pl.BlockSpec((D, 1), lambda i: (0, 0)),
      ],
      out_specs=[
          pl.BlockSpec((_BLK, 1), lambda i: (i, 0)),
          pl.BlockSpec((_BLK, D), lambda i: (i, 0)),
      ],
      out_shape=[
          jax.ShapeDtypeStruct((N, 1), jnp.float32),
          jax.ShapeDtypeStruct((N, D), jnp.float32),
      ],
  )(s_partials, u, degt, b_conv.reshape(1, D), w_pool.reshape(D, 1))

  pooled = _pool_kernel(
      batch, score2.reshape(N), hp, iidx2d, zer1).reshape(G, D)

  out = pl.pallas_call(
      _tcc_body,
      in_specs=[
          pl.BlockSpec((G, D), lambda: (0, 0)),
          pl.BlockSpec((D, D), lambda: (0, 0)),
          pl.BlockSpec((1, D), lambda: (0, 0)),
      ],
      out_specs=pl.BlockSpec((G, D), lambda: (0, 0)),
      out_shape=jax.ShapeDtypeStruct((G, D), jnp.float32),
  )(pooled, W_lin, b_lin.reshape(1, D))
  return out


# speedup vs baseline: 1.0795x; 1.0239x over previous
import functools

import jax
import jax.numpy as jnp
from jax import lax
from jax.experimental import pallas as pl
from jax.experimental.pallas import tpu as pltpu
from jax.experimental.pallas import tpu_sc as plsc

N = 10000
E = 320000
D = 128
G = 64
NC = 2
NS = 16
NW = NC * NS
WIN = 128
NWIN = E // WIN
WIN_BASE = NWIN // NW
WIN_REM = NWIN % NW
ROWBLK = 640
NZWIN = (N + WIN - 1) // WIN
ZWPT = 5

_mesh = functools.partial(
    plsc.VectorSubcoreMesh, core_axis_name="c", subcore_axis_name="s")


def _wid():
  return lax.axis_index("s") * NC + lax.axis_index("c")


def _vconsts():
  lane = lax.iota(jnp.int32, 16)
  zero16 = plsc.bitcast(lane * 0, jnp.float32)
  one16 = plsc.bitcast(lane * 0 + 0x3F800000, jnp.float32)
  return lane, zero16, one16


def _sload(ref, i):
  return ref[pl.ds(i, 16)][0]


@functools.partial(
    pl.kernel,
    out_type=jax.ShapeDtypeStruct((NC * N,), jnp.float32),
    mesh=_mesh(),
    scratch_types=[
        pltpu.VMEM((WIN,), jnp.int32),
        pltpu.VMEM((WIN,), jnp.int32),
        pltpu.VMEM((WIN,), jnp.float32),
        pltpu.VMEM((WIN,), jnp.float32),
        pltpu.VMEM((ROWBLK,), jnp.float32),
        pltpu.VMEM_SHARED((N,), jnp.float32),
        pltpu.SemaphoreType.DMA,
        pltpu.SemaphoreType.DMA,
    ],
)
def _deg_kernel(dst2d, iidx2d, ones_hbm, zer_hbm, out,
                didxA, didxB, ones_v, zer_v, stage, acc, semA, semB):
  cid = lax.axis_index("c")
  sid = lax.axis_index("s")
  wid = _wid()
  pltpu.sync_copy(ones_hbm, ones_v)
  pltpu.sync_copy(zer_hbm, zer_v)

  @pl.loop(0, ZWPT)
  def _(j):
    pltpu.sync_copy(iidx2d.at[sid * ZWPT + j], didxA)
    pltpu.sync_copy(zer_v, acc.at[didxA])

  plsc.subcore_barrier()

  base = wid * WIN_BASE + jnp.minimum(wid, WIN_REM)
  nwin = WIN_BASE + jnp.where(wid < WIN_REM, 1, 0)

  pltpu.sync_copy(dst2d.at[base], didxA)

  @pl.when(nwin > 1)
  def _():
    pltpu.async_copy(dst2d.at[base + 1], didxB, semB)

  @pl.loop(0, nwin, step=2)
  def _(w):
    pltpu.sync_copy(ones_v, acc.at[didxA], add=True)

    @pl.when(w + 2 < nwin)
    def _():
      pltpu.async_copy(dst2d.at[base + w + 2], didxA, semA)

    @pl.when(w + 1 < nwin)
    def _():
      pltpu.make_async_copy(dst2d.at[base + w + 1], didxB, semB).wait()
      pltpu.sync_copy(ones_v, acc.at[didxB], add=True)

      @pl.when(w + 3 < nwin)
      def _():
        pltpu.async_copy(dst2d.at[base + w + 3], didxB, semB)

    @pl.when(w + 2 < nwin)
    def _():
      pltpu.make_async_copy(dst2d.at[base + w + 2], didxA, semA).wait()

  plsc.subcore_barrier()
  st = jnp.minimum(sid * ROWBLK, N - ROWBLK)
  off = pl.multiple_of(cid * N + st, 16)
  pltpu.sync_copy(acc.at[pl.ds(st, ROWBLK)], stage)
  pltpu.sync_copy(stage, out.at[pl.ds(off, ROWBLK)])


@functools.partial(
    pl.kernel,
    out_type=jax.ShapeDtypeStruct((NC, N, D), jnp.float32),
    mesh=_mesh(),
    scratch_types=[
        pltpu.VMEM((WIN,), jnp.int32),
        pltpu.VMEM((WIN,), jnp.int32),
        pltpu.VMEM((WIN,), jnp.int32),
        pltpu.VMEM((WIN,), jnp.int32),
        pltpu.VMEM((WIN, D), jnp.float32),
        pltpu.VMEM((WIN, D), jnp.float32),
        pltpu.VMEM_SHARED((N, D), jnp.float32),
        pltpu.SemaphoreType.DMA,
        pltpu.SemaphoreType.DMA,
        pltpu.SemaphoreType.DMA,
        pltpu.SemaphoreType.DMA,
    ],
)
def _scatter_kernel(u_hbm, src2d, dst2d, iidx2d, zrows_hbm, out,
                    sidxA, didxA, sidxB, didxB, rowsA, rowsB, sacc,
                    semGA, semGB, semIA, semIB):
  cid = lax.axis_index("c")
  sid = lax.axis_index("s")
  wid = _wid()

  pltpu.sync_copy(iidx2d.at[sid * ZWPT], sidxA)
  pltpu.async_copy(zrows_hbm.at[sidxA], rowsA, semGA).wait()

  @pl.loop(0, ZWPT)
  def _(j):
    pltpu.sync_copy(iidx2d.at[sid * ZWPT + j], sidxA)
    pltpu.sync_copy(rowsA, sacc.at[sidxA])

  plsc.subcore_barrier()

  base = wid * WIN_BASE + jnp.minimum(wid, WIN_REM)
  nwin = WIN_BASE + jnp.where(wid < WIN_REM, 1, 0)

  def start_idx(win, si, di, sem):
    pltpu.async_copy(src2d.at[win], si, sem)
    pltpu.async_copy(dst2d.at[win], di, sem)

  def wait_idx(win, si, di, sem):
    pltpu.make_async_copy(src2d.at[win], si, sem).wait()
    pltpu.make_async_copy(dst2d.at[win], di, sem).wait()

  def start_g(si, rows, sem):
    pltpu.async_copy(u_hbm.at[si], rows, sem)

  def wait_g(si, rows, sem):
    pltpu.make_async_copy(u_hbm.at[si], rows, sem).wait()

  pltpu.sync_copy(src2d.at[base], sidxA)
  pltpu.sync_copy(dst2d.at[base], didxA)
  start_g(sidxA, rowsA, semGA)

  @pl.when(nwin > 1)
  def _():
    pltpu.sync_copy(src2d.at[base + 1], sidxB)
    pltpu.sync_copy(dst2d.at[base + 1], didxB)
    start_g(sidxB, rowsB, semGB)

  @pl.loop(0, nwin, step=2)
  def _(w):
    wait_g(sidxA, rowsA, semGA)
    pltpu.sync_copy(rowsA, sacc.at[didxA], add=True)

    @pl.when(w + 2 < nwin)
    def _():
      start_idx(base + w + 2, sidxA, didxA, semIA)
      wait_idx(base + w + 2, sidxA, didxA, semIA)
      start_g(sidxA, rowsA, semGA)

    @pl.when(w + 1 < nwin)
    def _():
      wait_g(sidxB, rowsB, semGB)
      pltpu.sync_copy(rowsB, sacc.at[didxB], add=True)

      @pl.when(w + 3 < nwin)
      def _():
        start_idx(base + w + 3, sidxB, didxB, semIB)
        wait_idx(base + w + 3, sidxB, didxB, semIB)
        start_g(sidxB, rowsB, semGB)

  plsc.subcore_barrier()
  st = jnp.minimum(sid * ROWBLK, N - ROWBLK)
  pltpu.sync_copy(sacc.at[pl.ds(st, ROWBLK)], out.at[cid, pl.ds(st, ROWBLK)])


NPAD = N + 16


@functools.partial(
    pl.kernel,
    out_type=jax.ShapeDtypeStruct((G * D,), jnp.float32),
    mesh=_mesh(),
    scratch_types=[
        pltpu.VMEM((NPAD,), jnp.int32),
        pltpu.VMEM((NPAD,), jnp.float32),
        pltpu.VMEM((NPAD,), jnp.int32),
        pltpu.VMEM((WIN,), jnp.int32),
        pltpu.VMEM((WIN, D), jnp.float32),
        pltpu.VMEM((D,), jnp.float32),
        pltpu.SemaphoreType.DMA,
    ],
)
def _pool_kernel(batch_hbm, score_hbm, hp_hbm, iidx2d, zer_hbm, out,
                 bb, sb, fl, ridx, hb, ab, sem):
  wid = _wid()
  lane, _, _ = _vconsts()
  pltpu.sync_copy(batch_hbm, bb.at[pl.ds(0, N)])
  pltpu.sync_copy(score_hbm, sb.at[pl.ds(0, N)])
  bb[pl.ds(N, 16)] = lane * 0 + G
  pltpu.sync_copy(zer_hbm.at[pl.ds(0, 16)], sb.at[pl.ds(N, 16)])

  def lower_bound(val):
    def body(_, lohi):
      lo, hi = lohi
      mid = (lo + hi) // 2
      p = _sload(bb, mid) < val
      return jnp.where(p, mid + 1, lo), jnp.where(p, hi, mid)
    lo, _ = lax.fori_loop(0, 14, body, (jnp.int32(0), jnp.int32(N)))
    return lo

  for gg in range(2):
    g = wid * 2 + gg
    st = lower_bound(g)
    en = lower_bound(g + 1)
    c = en - st
    kk = (c + 1) // 2
    pltpu.sync_copy(zer_hbm, ab)

    @pl.when(c > 0)
    def _():
      nchunk = (c + 15) // 16

      @pl.loop(0, nchunk)
      def _(ic):
        ibase = st + ic * 16
        iv = ibase + lane
        si = sb[pl.ds(ibase, 16)]

        def jbody(j, cnt):
          sj = _sload(sb, j)
          hit = (sj > si) | ((sj == si) & (j < iv))
          return cnt + jnp.where(hit, 1, 0)

        cnt = lax.fori_loop(st, en, jbody, lane * 0)
        selv = (cnt < kk) & (iv < en)
        fl[pl.ds(ic * 16, 16)] = jnp.where(selv, 1, 0)

      wb = st // WIN
      nhw = (en - wb * WIN + WIN - 1) // WIN

      @pl.loop(0, nhw)
      def _(wj):
        pltpu.sync_copy(iidx2d.at[wb + wj], ridx)
        pltpu.async_copy(hp_hbm.at[ridx], hb, sem).wait()

        @pl.loop(0, WIN)
        def _(r):
          gr = (wb + wj) * WIN + r
          ok = (gr >= st) & (gr < en)

          @pl.when(ok)
          def _():
            @pl.when(_sload(fl, gr - st) > 0)
            def _():
              for v in range(D // 16):
                sl = pl.ds(v * 16, 16)
                ab[sl] = jnp.maximum(ab[sl], hb[r, sl])

    off = pl.multiple_of(g * D, 16)
    pltpu.sync_copy(ab, out.at[pl.ds(off, D)])


_BLK = 400
_DOT = dict(preferred_element_type=jnp.float32, precision=lax.Precision.HIGHEST)


def _tca_body(x_ref, w_ref, degt_ref, u_ref):
  h = jnp.dot(x_ref[...], w_ref[...], **_DOT)
  deg = degt_ref[:, 0:1] + degt_ref[:, 1:2] + 1.0
  dinv = jnp.where(deg > 0, lax.rsqrt(deg), 0.0)
  u_ref[...] = h * dinv


def _tcc_body(p_ref, wl_ref, bl_ref, o_ref):
  o_ref[...] = jnp.dot(p_ref[...], wl_ref[...], **_DOT) + bl_ref[...]


def _tcb_body(s_ref, u_ref, degt_ref, b_ref, w_ref, score_ref, hp_ref):
  s_sum = s_ref[0] + s_ref[1]
  deg = degt_ref[:, 0:1] + degt_ref[:, 1:2] + 1.0
  dinv = jnp.where(deg > 0, lax.rsqrt(deg), 0.0)
  htot = dinv * (s_sum + u_ref[...]) + b_ref[...]
  w = w_ref[...]
  nw = jnp.sqrt(jnp.sum(w * w))
  z = jnp.dot(htot, w, **_DOT) / nw
  sc = jnp.tanh(z)
  score_ref[...] = sc
  hp_ref[...] = jnp.maximum(htot * sc, 0.0)


def kernel(x, edge_index, batch, W_conv, b_conv, w_pool, W_lin, b_lin):
  src2d = edge_index[0].reshape(NWIN, WIN)
  dst2d = edge_index[1].reshape(NWIN, WIN)
  flat = jnp.minimum(jnp.arange((NZWIN + 1) * WIN, dtype=jnp.int32), N - 1)
  iidx2d = jnp.concatenate(
      [flat, jnp.arange(WIN, dtype=jnp.int32)]).reshape(NZWIN + 2, WIN)
  zrows = jnp.zeros((N, D), jnp.float32)
  ones1 = jnp.ones((WIN,), jnp.float32)
  zer1 = jnp.zeros((WIN,), jnp.float32)

  degp = _deg_kernel(dst2d, iidx2d, ones1, zer1).reshape(NC, N)
  degt = degp.T

  u = pl.pallas_call(
      _tca_body,
      grid=(N // _BLK,),
      in_specs=[
          pl.BlockSpec((_BLK, D), lambda i: (i, 0)),
          pl.BlockSpec((D, D), lambda i: (0, 0)),
          pl.BlockSpec((_BLK, 2), lambda i: (i, 0)),
      ],
      out_specs=pl.BlockSpec((_BLK, D), lambda i: (i, 0)),
      out_shape=jax.ShapeDtypeStruct((N, D), jnp.float32),
  )(x, W_conv, degt)

  s_partials = _scatter_kernel(u, src2d, dst2d, iidx2d, zrows)

  score2, hp = pl.pallas_call(
      _tcb_body,
      grid=(N // _BLK,),
      in_specs=[
          pl.BlockSpec((NC, _BLK, D), lambda i: (0, i, 0)),
          pl.BlockSpec((_BLK, D), lambda i: (i, 0)),
          pl.BlockSpec((_BLK, 2), lambda i: (i, 0)),
          pl.BlockSpec((1, D), lambda i: (0, 0)),
          pl.BlockSpec((D, 1), lambda i: (0, 0)),
      ],
      out_specs=[
          pl.BlockSpec((_BLK, 1), lambda i: (i, 0)),
          pl.BlockSpec((_BLK, D), lambda i: (i, 0)),
      ],
      out_shape=[
          jax.ShapeDtypeStruct((N, 1), jnp.float32),
          jax.ShapeDtypeStruct((N, D), jnp.float32),
      ],
  )(s_partials, u, degt, b_conv.reshape(1, D), w_pool.reshape(D, 1))

  pooled = _pool_kernel(
      batch, score2.reshape(N), hp, iidx2d, zer1).reshape(G, D)

  out = pl.pallas_call(
      _tcc_body,
      in_specs=[
          pl.BlockSpec((G, D), lambda: (0, 0)),
          pl.BlockSpec((D, D), lambda: (0, 0)),
          pl.BlockSpec((1, D), lambda: (0, 0)),
      ],
      out_specs=pl.BlockSpec((G, D), lambda: (0, 0)),
      out_shape=jax.ShapeDtypeStruct((G, D), jnp.float32),
  )(pooled, W_lin, b_lin.reshape(1, D))
  return out
